# baseline (speedup 1.0000x reference)
import functools

import jax
import jax.numpy as jnp
from jax import lax
from jax.experimental import pallas as pl
from jax.experimental.pallas import tpu as pltpu
from jax.experimental.pallas import tpu_sc as plsc

_BQ = 1024
_BN = 2048

_NC, _NS, _L = 2, 16, 16
_NW = _NC * _NS


_IDBITS = 7
_IDMASK = (1 << _IDBITS) - 1
_GCHUNKS = 8


def _prep_body(k_ref, q_ref, kaug_ref, qaug_ref, *, n, bn):
    i = pl.program_id(0)
    kb = k_ref[...]
    rows = i * bn + lax.broadcasted_iota(jnp.int32, (bn, 1), 0)
    valid = rows < n
    kb = jnp.where(valid, kb, 0.0)
    ksq = jnp.sum(kb * kb, axis=1, keepdims=True)
    ksq = jnp.where(valid, ksq, 1e9)
    ksq_hi = ksq.astype(jnp.bfloat16).astype(jnp.float32)
    ksq_lo = ksq - ksq_hi
    ones = jnp.ones_like(ksq)
    kaug = jnp.concatenate([kb, ksq_hi, ksq_lo, ones, ones], axis=1)
    kaug_ref[...] = kaug.astype(jnp.bfloat16)
    qb = q_ref[...]
    qsq = jnp.sum(qb * qb, axis=1, keepdims=True)
    qsq_hi = qsq.astype(jnp.bfloat16).astype(jnp.float32)
    qsq_lo = qsq - qsq_hi
    ones_q = jnp.ones_like(qsq)
    qaug = jnp.concatenate([qb * -2.0, ones_q, ones_q, qsq_hi, qsq_lo],
                           axis=1)
    qaug_ref[...] = qaug.astype(jnp.bfloat16)


def _prep(inputs, keys, npad):
    n, d = keys.shape
    q = inputs.shape[0]
    qb_last = q // _BN - 1
    return pl.pallas_call(
        functools.partial(_prep_body, n=n, bn=_BN),
        grid=(npad // _BN,),
        in_specs=[
            pl.BlockSpec((_BN, d), lambda i: (i, 0)),
            pl.BlockSpec((_BN, d), lambda i: (jnp.minimum(i, qb_last), 0)),
        ],
        out_specs=[
            pl.BlockSpec((_BN, d + 4), lambda i: (i, 0)),
            pl.BlockSpec((_BN, d + 4), lambda i: (jnp.minimum(i, qb_last), 0)),
        ],
        out_shape=[
            jax.ShapeDtypeStruct((npad, d + 4), jnp.bfloat16),
            jax.ShapeDtypeStruct((q, d + 4), jnp.bfloat16),
        ],
    )(keys, inputs)


def _nn_body(q_ref, k_ref, out_ref, acc_ref, *, bn):
    inn = pl.program_id(1)
    nn = pl.num_programs(1)
    nchunks = bn // 128

    d2 = lax.dot_general(q_ref[...], k_ref[...], (((1,), (1,)), ((), ())),
                         preferred_element_type=jnp.float32)

    ngroups = nchunks // _GCHUNKS
    base = inn * ngroups
    m = None
    for g in range(ngroups):
        mg = None
        for c in range(_GCHUNKS):
            cc = g * _GCHUNKS + c
            sl = d2[:, cc * 128:(cc + 1) * 128]
            mg = sl if mg is None else jnp.minimum(mg, sl)
        pc = (lax.bitcast_convert_type(mg, jnp.int32) & jnp.int32(~_IDMASK)
              ) | (base + g)
        pf = lax.bitcast_convert_type(pc, jnp.float32)
        m = pf if m is None else jnp.minimum(m, pf)

    @pl.when(inn == 0)
    def _():
        acc_ref[...] = m

    @pl.when(inn > 0)
    def _():
        acc_ref[...] = jnp.minimum(acc_ref[...], m)

    @pl.when(inn == nn - 1)
    def _():
        merged = acc_ref[...]
        fmin = jnp.min(merged, axis=1, keepdims=True)
        lane128 = lax.broadcasted_iota(jnp.int32, merged.shape, 1)
        lane = jnp.min(jnp.where(merged == fmin, lane128, jnp.int32(127)),
                       axis=1, keepdims=True)
        idp = lax.bitcast_convert_type(fmin, jnp.int32) & jnp.int32(_IDMASK)
        out_ref[...] = (idp * (_GCHUNKS * 128) + lane)[:, 0]


def _nn_argmin(inputs, keys):
    q, d = inputs.shape
    n = keys.shape[0]
    npad = ((n + _BN - 1) // _BN) * _BN
    kaug, qaug = _prep(inputs, keys, npad)
    grid = (q // _BQ, npad // _BN)
    return pl.pallas_call(
        functools.partial(_nn_body, bn=_BN),
        grid=grid,
        in_specs=[
            pl.BlockSpec((_BQ, d + 4), lambda iq, inn: (iq, 0)),
            pl.BlockSpec((_BN, d + 4), lambda iq, inn: (inn, 0)),
        ],
        out_specs=pl.BlockSpec((_BQ,), lambda iq, inn: (iq,)),
        out_shape=jax.ShapeDtypeStruct((q,), jnp.int32),
        scratch_shapes=[
            pltpu.VMEM((_BQ, 128), jnp.float32),
        ],
        compiler_params=pltpu.CompilerParams(
            dimension_semantics=("parallel", "arbitrary")),
    )(qaug, kaug)


def _sc_finalize(inputs, keys, w_flat, cand):
    q = cand.shape[0]
    n, d = keys.shape
    bpw = q // _NW
    ngrp = bpw // _L
    mesh = plsc.VectorSubcoreMesh(core_axis_name="c", subcore_axis_name="s")

    @functools.partial(
        pl.kernel,
        out_type=jax.ShapeDtypeStruct((2, q), jnp.float32),
        mesh=mesh,
        scratch_types=[
            pltpu.VMEM((bpw,), jnp.int32),
            pltpu.VMEM((_GCHUNKS, bpw), jnp.int32),
            pltpu.VMEM((bpw, d), jnp.float32),
            pltpu.VMEM((_GCHUNKS, bpw, d), jnp.float32),
            pltpu.VMEM((bpw,), jnp.int32),
            pltpu.VMEM((bpw,), jnp.float32),
            pltpu.VMEM((bpw,), jnp.float32),
            pltpu.VMEM((bpw,), jnp.float32),
            pltpu.SemaphoreType.DMA,
            pltpu.SemaphoreType.DMA,
        ],
        compiler_params=pltpu.CompilerParams(needs_layout_passes=False,
                                             use_tc_tiling_on_sc=False),
    )
    def k(q_hbm, keys_hbm, w_hbm, cand_hbm, out_hbm, cand_v, idx8_v,
          qrows_v, krows_v, idxw_v, wv_v, y0_v, y1_v, sem, semw):
        wid = lax.axis_index("s") * _NC + lax.axis_index("c")
        base = wid * bpw
        pltpu.sync_copy(cand_hbm.at[pl.ds(base, bpw)], cand_v)
        pltpu.sync_copy(q_hbm.at[pl.ds(base, bpw)], qrows_v)
        iota16 = lax.iota(jnp.int32, _L)
        for i in range(ngrp):
            c16 = cand_v[pl.ds(i * _L, _L)]
            for c in range(_GCHUNKS):
                idx8_v[c, pl.ds(i * _L, _L)] = jnp.minimum(
                    c16 + (c * 128), n - 1)
        copies = [pltpu.async_copy(keys_hbm.at[idx8_v.at[c]], krows_v.at[c],
                                   sem)
                  for c in range(_GCHUNKS)]
        for cp in copies:
            cp.wait()
        for i in range(ngrp):
            j16 = iota16 + (i * _L)
            cvecs = [jnp.full((_L,), c, jnp.int32) for c in range(_GCHUNKS)]

            def body(dim, accs):
                dimv = jnp.full((_L,), dim, jnp.int32)
                qv = plsc.load_gather(qrows_v, [j16, dimv])
                out = []
                for c in range(_GCHUNKS):
                    kv = plsc.load_gather(krows_v, [cvecs[c], j16, dimv])
                    df = qv - kv
                    out.append(accs[c] + df * df)
                return tuple(out)

            accs = lax.fori_loop(
                0, d, body,
                tuple(jnp.zeros((_L,), jnp.float32)
                      for _ in range(_GCHUNKS)))
            best = accs[0]
            bestc = jnp.zeros((_L,), jnp.int32)
            for c in range(1, _GCHUNKS):
                lt = accs[c] < best
                best = jnp.where(lt, accs[c], best)
                bestc = jnp.where(lt, cvecs[c], bestc)
            idxw_v[pl.ds(i * _L, _L)] = jnp.minimum(
                cand_v[pl.ds(i * _L, _L)] + bestc * 128, n - 1)
        pltpu.async_copy(w_hbm.at[idxw_v], wv_v, semw).wait()
        for i in range(ngrp):
            x = wv_v[pl.ds(i * _L, _L)]
            s = 1.0 / (1.0 + jnp.exp(-x))
            y0_v[pl.ds(i * _L, _L)] = s
            y1_v[pl.ds(i * _L, _L)] = 1.0 - s
        pltpu.sync_copy(y0_v, out_hbm.at[0, pl.ds(base, bpw)])
        pltpu.sync_copy(y1_v, out_hbm.at[1, pl.ds(base, bpw)])

    return k(inputs, keys, w_flat, cand)


def kernel(inputs, keys, w):
    cand = _nn_argmin(inputs, keys)
    y01 = _sc_finalize(inputs, keys, w.reshape(-1), cand)
    return y01.T


# speedup vs baseline: 1.2333x; 1.2333x over previous
import functools

import jax
import jax.numpy as jnp
from jax import lax
from jax.experimental import pallas as pl
from jax.experimental.pallas import tpu as pltpu
from jax.experimental.pallas import tpu_sc as plsc

_BQ = 1024
_BN = 2048

_NC, _NS, _L = 2, 16, 16
_NW = _NC * _NS


_IDBITS = 10
_IDMASK = (1 << _IDBITS) - 1


def _prep_body(k_ref, q_ref, kaug_ref, qaug_ref, *, n, bn):
    i = pl.program_id(0)
    kb = k_ref[...]
    rows = i * bn + lax.broadcasted_iota(jnp.int32, (bn, 1), 0)
    valid = rows < n
    kb = jnp.where(valid, kb, 0.0)
    ksq = jnp.sum(kb * kb, axis=1, keepdims=True)
    ksq = jnp.where(valid, ksq, 1e9)
    ksq_hi = ksq.astype(jnp.bfloat16).astype(jnp.float32)
    ksq_lo = ksq - ksq_hi
    ones = jnp.ones_like(ksq)
    kaug = jnp.concatenate([kb, ksq_hi, ksq_lo, ones, ones], axis=1)
    kaug_ref[...] = kaug.astype(jnp.bfloat16)
    qb = q_ref[...]
    qsq = jnp.sum(qb * qb, axis=1, keepdims=True)
    qsq_hi = qsq.astype(jnp.bfloat16).astype(jnp.float32)
    qsq_lo = qsq - qsq_hi
    ones_q = jnp.ones_like(qsq)
    qaug = jnp.concatenate([qb * -2.0, ones_q, ones_q, qsq_hi, qsq_lo],
                           axis=1)
    qaug_ref[...] = qaug.astype(jnp.bfloat16)


def _prep(inputs, keys, npad):
    n, d = keys.shape
    q = inputs.shape[0]
    qb_last = q // _BN - 1
    return pl.pallas_call(
        functools.partial(_prep_body, n=n, bn=_BN),
        grid=(npad // _BN,),
        in_specs=[
            pl.BlockSpec((_BN, d), lambda i: (i, 0)),
            pl.BlockSpec((_BN, d), lambda i: (jnp.minimum(i, qb_last), 0)),
        ],
        out_specs=[
            pl.BlockSpec((_BN, d + 4), lambda i: (i, 0)),
            pl.BlockSpec((_BN, d + 4), lambda i: (jnp.minimum(i, qb_last), 0)),
        ],
        out_shape=[
            jax.ShapeDtypeStruct((npad, d + 4), jnp.bfloat16),
            jax.ShapeDtypeStruct((q, d + 4), jnp.bfloat16),
        ],
    )(keys, inputs)


def _nn_body(q_ref, k_ref, out_ref, acc_ref, *, bn):
    inn = pl.program_id(1)
    nn = pl.num_programs(1)
    nchunks = bn // 128

    d2 = lax.dot_general(q_ref[...], k_ref[...], (((1,), (1,)), ((), ())),
                         preferred_element_type=jnp.float32)

    bitsm = lax.bitcast_convert_type(d2, jnp.int32) & jnp.int32(~_IDMASK)
    base = inn * nchunks
    packed = []
    for c in range(nchunks):
        pc = bitsm[:, c * 128:(c + 1) * 128] | (base + c)
        packed.append(lax.bitcast_convert_type(pc, jnp.float32))
    while len(packed) > 1:
        packed = [jnp.minimum(packed[i], packed[i + 1])
                  for i in range(0, len(packed) - 1, 2)] + (
                      [packed[-1]] if len(packed) % 2 else [])
    m = packed[0]

    @pl.when(inn == 0)
    def _():
        acc_ref[...] = m

    @pl.when(inn > 0)
    def _():
        acc_ref[...] = jnp.minimum(acc_ref[...], m)

    @pl.when(inn == nn - 1)
    def _():
        merged = acc_ref[...]
        fmin = jnp.min(merged, axis=1, keepdims=True)
        lane128 = lax.broadcasted_iota(jnp.int32, merged.shape, 1)
        lane = jnp.min(jnp.where(merged == fmin, lane128, jnp.int32(127)),
                       axis=1, keepdims=True)
        idp = lax.bitcast_convert_type(fmin, jnp.int32) & jnp.int32(_IDMASK)
        out_ref[...] = (idp * 128 + lane)[:, 0]


def _nn_argmin(inputs, keys):
    q, d = inputs.shape
    n = keys.shape[0]
    npad = ((n + _BN - 1) // _BN) * _BN
    kaug, qaug = _prep(inputs, keys, npad)
    grid = (q // _BQ, npad // _BN)
    return pl.pallas_call(
        functools.partial(_nn_body, bn=_BN),
        grid=grid,
        in_specs=[
            pl.BlockSpec((_BQ, d + 4), lambda iq, inn: (iq, 0)),
            pl.BlockSpec((_BN, d + 4), lambda iq, inn: (inn, 0)),
        ],
        out_specs=pl.BlockSpec((_BQ,), lambda iq, inn: (iq,)),
        out_shape=jax.ShapeDtypeStruct((q,), jnp.int32),
        scratch_shapes=[
            pltpu.VMEM((_BQ, 128), jnp.float32),
        ],
        compiler_params=pltpu.CompilerParams(
            dimension_semantics=("parallel", "arbitrary")),
    )(qaug, kaug)


def _gather_sigmoid(w_flat, idx):
    q = idx.shape[0]
    bpw = q // _NW
    mesh = plsc.VectorSubcoreMesh(core_axis_name="c", subcore_axis_name="s")

    @functools.partial(
        pl.kernel,
        out_type=jax.ShapeDtypeStruct((2, q), jnp.float32),
        mesh=mesh,
        scratch_types=[
            pltpu.VMEM((bpw,), jnp.int32),
            pltpu.VMEM((bpw,), jnp.float32),
            pltpu.VMEM((bpw,), jnp.float32),
            pltpu.VMEM((bpw,), jnp.float32),
            pltpu.SemaphoreType.DMA,
        ],
    )
    def k(w_hbm, idx_hbm, out_hbm, idx_v, val_v, y0_v, y1_v, sem):
        wid = lax.axis_index("s") * _NC + lax.axis_index("c")
        base = wid * bpw
        pltpu.sync_copy(idx_hbm.at[pl.ds(base, bpw)], idx_v)
        pltpu.async_copy(w_hbm.at[idx_v], val_v, sem).wait()
        for i in range(bpw // _L):
            x = val_v[pl.ds(i * _L, _L)]
            s = 1.0 / (1.0 + jnp.exp(-x))
            y0_v[pl.ds(i * _L, _L)] = s
            y1_v[pl.ds(i * _L, _L)] = 1.0 - s
        pltpu.sync_copy(y0_v, out_hbm.at[0, pl.ds(base, bpw)])
        pltpu.sync_copy(y1_v, out_hbm.at[1, pl.ds(base, bpw)])

    return k(w_flat, idx)


def kernel(inputs, keys, w):
    idx = _nn_argmin(inputs, keys)
    y01 = _gather_sigmoid(w.reshape(-1), idx)
    return y01.T


# speedup vs baseline: 1.3056x; 1.0585x over previous
import functools

import jax
import jax.numpy as jnp
from jax import lax
from jax.experimental import pallas as pl
from jax.experimental.pallas import tpu as pltpu
from jax.experimental.pallas import tpu_sc as plsc

_BQ = 2048
_BN = 2048

_NC, _NS, _L = 2, 16, 16
_NW = _NC * _NS


_IDBITS = 10
_IDMASK = (1 << _IDBITS) - 1


def _prep_body(k_ref, q_ref, kaug_ref, qaug_ref, *, n, bn):
    i = pl.program_id(0)
    kb = k_ref[...]
    rows = i * bn + lax.broadcasted_iota(jnp.int32, (bn, 1), 0)
    valid = rows < n
    kb = jnp.where(valid, kb, 0.0)
    ones_col = jnp.ones((kb.shape[1], 1), jnp.float32)
    ksq = lax.dot_general(kb * kb, ones_col, (((1,), (0,)), ((), ())),
                          preferred_element_type=jnp.float32)
    ksq = jnp.where(valid, ksq, 1e9)
    ksq_hi = ksq.astype(jnp.bfloat16)
    ksq_lo = (ksq - ksq_hi.astype(jnp.float32)).astype(jnp.bfloat16)
    ones = jnp.ones_like(ksq_hi)
    kaug_ref[...] = jnp.concatenate(
        [kb.astype(jnp.bfloat16), ksq_hi, ksq_lo, ones, ones], axis=1)
    qb = q_ref[...]
    qsq = lax.dot_general(qb * qb, ones_col, (((1,), (0,)), ((), ())),
                          preferred_element_type=jnp.float32)
    qsq_hi = qsq.astype(jnp.bfloat16)
    qsq_lo = (qsq - qsq_hi.astype(jnp.float32)).astype(jnp.bfloat16)
    ones_q = jnp.ones_like(qsq_hi)
    qaug_ref[...] = jnp.concatenate(
        [(qb * -2.0).astype(jnp.bfloat16), ones_q, ones_q, qsq_hi, qsq_lo],
        axis=1)


def _prep(inputs, keys, npad):
    n, d = keys.shape
    q = inputs.shape[0]
    qb_last = q // _BN - 1
    return pl.pallas_call(
        functools.partial(_prep_body, n=n, bn=_BN),
        grid=(npad // _BN,),
        in_specs=[
            pl.BlockSpec((_BN, d), lambda i: (i, 0)),
            pl.BlockSpec((_BN, d), lambda i: (jnp.minimum(i, qb_last), 0)),
        ],
        out_specs=[
            pl.BlockSpec((_BN, d + 4), lambda i: (i, 0)),
            pl.BlockSpec((_BN, d + 4), lambda i: (jnp.minimum(i, qb_last), 0)),
        ],
        out_shape=[
            jax.ShapeDtypeStruct((npad, d + 4), jnp.bfloat16),
            jax.ShapeDtypeStruct((q, d + 4), jnp.bfloat16),
        ],
    )(keys, inputs)


def _nn_body(q_ref, k_ref, out_ref, acc_ref, *, bn):
    inn = pl.program_id(1)
    nn = pl.num_programs(1)
    nchunks = bn // 128

    d2 = lax.dot_general(q_ref[...], k_ref[...], (((1,), (1,)), ((), ())),
                         preferred_element_type=jnp.float32)

    bitsm = lax.bitcast_convert_type(d2, jnp.int32) & jnp.int32(~_IDMASK)
    base = inn * nchunks
    packed = []
    for c in range(nchunks):
        pc = bitsm[:, c * 128:(c + 1) * 128] | (base + c)
        packed.append(lax.bitcast_convert_type(pc, jnp.float32))
    while len(packed) > 1:
        packed = [jnp.minimum(packed[i], packed[i + 1])
                  for i in range(0, len(packed) - 1, 2)] + (
                      [packed[-1]] if len(packed) % 2 else [])
    m = packed[0]

    @pl.when(inn == 0)
    def _():
        acc_ref[...] = m

    @pl.when(inn > 0)
    def _():
        acc_ref[...] = jnp.minimum(acc_ref[...], m)

    @pl.when(inn == nn - 1)
    def _():
        merged = acc_ref[...]
        fmin = jnp.min(merged, axis=1, keepdims=True)
        lane128 = lax.broadcasted_iota(jnp.int32, merged.shape, 1)
        lane = jnp.min(jnp.where(merged == fmin, lane128, jnp.int32(127)),
                       axis=1, keepdims=True)
        idp = lax.bitcast_convert_type(fmin, jnp.int32) & jnp.int32(_IDMASK)
        out_ref[...] = (idp * 128 + lane)[:, 0]


def _nn_argmin(inputs, keys):
    q, d = inputs.shape
    n = keys.shape[0]
    npad = ((n + _BN - 1) // _BN) * _BN
    kaug, qaug = _prep(inputs, keys, npad)
    grid = (q // _BQ, npad // _BN)
    return pl.pallas_call(
        functools.partial(_nn_body, bn=_BN),
        grid=grid,
        in_specs=[
            pl.BlockSpec((_BQ, d + 4), lambda iq, inn: (iq, 0)),
            pl.BlockSpec((_BN, d + 4), lambda iq, inn: (inn, 0)),
        ],
        out_specs=pl.BlockSpec((_BQ,), lambda iq, inn: (iq,)),
        out_shape=jax.ShapeDtypeStruct((q,), jnp.int32),
        scratch_shapes=[
            pltpu.VMEM((_BQ, 128), jnp.float32),
        ],
        compiler_params=pltpu.CompilerParams(
            dimension_semantics=("parallel", "arbitrary")),
    )(qaug, kaug)


def _gather_sigmoid(w_flat, idx):
    q = idx.shape[0]
    bpw = q // _NW
    mesh = plsc.VectorSubcoreMesh(core_axis_name="c", subcore_axis_name="s")

    @functools.partial(
        pl.kernel,
        out_type=jax.ShapeDtypeStruct((2, q), jnp.float32),
        mesh=mesh,
        scratch_types=[
            pltpu.VMEM((bpw,), jnp.int32),
            pltpu.VMEM((bpw,), jnp.float32),
            pltpu.VMEM((bpw,), jnp.float32),
            pltpu.VMEM((bpw,), jnp.float32),
            pltpu.SemaphoreType.DMA,
        ],
    )
    def k(w_hbm, idx_hbm, out_hbm, idx_v, val_v, y0_v, y1_v, sem):
        wid = lax.axis_index("s") * _NC + lax.axis_index("c")
        base = wid * bpw
        pltpu.sync_copy(idx_hbm.at[pl.ds(base, bpw)], idx_v)
        pltpu.async_copy(w_hbm.at[idx_v], val_v, sem).wait()
        for i in range(bpw // _L):
            x = val_v[pl.ds(i * _L, _L)]
            s = 1.0 / (1.0 + jnp.exp(-x))
            y0_v[pl.ds(i * _L, _L)] = s
            y1_v[pl.ds(i * _L, _L)] = 1.0 - s
        pltpu.sync_copy(y0_v, out_hbm.at[0, pl.ds(base, bpw)])
        pltpu.sync_copy(y1_v, out_hbm.at[1, pl.ds(base, bpw)])

    return k(w_flat, idx)


def kernel(inputs, keys, w):
    idx = _nn_argmin(inputs, keys)
    y01 = _gather_sigmoid(w.reshape(-1), idx)
    return y01.T


# speedup vs baseline: 1.3264x; 1.0160x over previous
import functools

import jax
import jax.numpy as jnp
from jax import lax
from jax.experimental import pallas as pl
from jax.experimental.pallas import tpu as pltpu
from jax.experimental.pallas import tpu_sc as plsc

_BQ = 2048
_BN = 2048

_NC, _NS, _L = 2, 16, 16
_NW = _NC * _NS


_IDBITS = 10
_IDMASK = (1 << _IDBITS) - 1


def _prep_body(k_ref, q_ref, kaug_ref, qaug_ref, *, n, bn):
    i = pl.program_id(0)
    kbt = jnp.transpose(k_ref[...], (1, 0))
    cols = i * bn + lax.broadcasted_iota(jnp.int32, (1, bn), 1)
    valid = cols < n
    kbt = jnp.where(valid, kbt, 0.0)
    ones_row = jnp.ones((1, kbt.shape[0]), jnp.float32)
    ksq = lax.dot_general(ones_row, kbt * kbt, (((1,), (0,)), ((), ())),
                          preferred_element_type=jnp.float32)
    ksq = jnp.where(valid, ksq, 1e9)
    ksq_hi = ksq.astype(jnp.bfloat16)
    ksq_lo = (ksq - ksq_hi.astype(jnp.float32)).astype(jnp.bfloat16)
    ones = jnp.ones_like(ksq_hi)
    kaug_ref[...] = jnp.concatenate(
        [kbt.astype(jnp.bfloat16), ksq_hi, ksq_lo, ones, ones], axis=0)
    qbt = jnp.transpose(q_ref[...], (1, 0))
    qsq = lax.dot_general(ones_row, qbt * qbt, (((1,), (0,)), ((), ())),
                          preferred_element_type=jnp.float32)
    qsq_hi = qsq.astype(jnp.bfloat16)
    qsq_lo = (qsq - qsq_hi.astype(jnp.float32)).astype(jnp.bfloat16)
    ones_q = jnp.ones_like(qsq_hi)
    qaug_ref[...] = jnp.concatenate(
        [(qbt * -2.0).astype(jnp.bfloat16), ones_q, ones_q, qsq_hi, qsq_lo],
        axis=0)


def _prep(inputs, keys, npad):
    n, d = keys.shape
    q = inputs.shape[0]
    qb_last = q // _BN - 1
    return pl.pallas_call(
        functools.partial(_prep_body, n=n, bn=_BN),
        grid=(npad // _BN,),
        in_specs=[
            pl.BlockSpec((_BN, d), lambda i: (i, 0)),
            pl.BlockSpec((_BN, d), lambda i: (jnp.minimum(i, qb_last), 0)),
        ],
        out_specs=[
            pl.BlockSpec((d + 4, _BN), lambda i: (0, i)),
            pl.BlockSpec((d + 4, _BN), lambda i: (0, jnp.minimum(i, qb_last))),
        ],
        out_shape=[
            jax.ShapeDtypeStruct((d + 4, npad), jnp.bfloat16),
            jax.ShapeDtypeStruct((d + 4, q), jnp.bfloat16),
        ],
    )(keys, inputs)


def _nn_body(q_ref, k_ref, out_ref, acc_ref, *, bn):
    inn = pl.program_id(1)
    nn = pl.num_programs(1)
    nchunks = bn // 128

    d2 = lax.dot_general(q_ref[...], k_ref[...], (((0,), (0,)), ((), ())),
                         preferred_element_type=jnp.float32)

    bitsm = lax.bitcast_convert_type(d2, jnp.int32) & jnp.int32(~_IDMASK)
    base = inn * nchunks
    packed = []
    for c in range(nchunks):
        pc = bitsm[:, c * 128:(c + 1) * 128] | (base + c)
        packed.append(lax.bitcast_convert_type(pc, jnp.float32))
    while len(packed) > 1:
        packed = [jnp.minimum(packed[i], packed[i + 1])
                  for i in range(0, len(packed) - 1, 2)] + (
                      [packed[-1]] if len(packed) % 2 else [])
    m = packed[0]

    @pl.when(inn == 0)
    def _():
        acc_ref[...] = m

    @pl.when(inn > 0)
    def _():
        acc_ref[...] = jnp.minimum(acc_ref[...], m)

    @pl.when(inn == nn - 1)
    def _():
        merged = acc_ref[...]
        fmin = jnp.min(merged, axis=1, keepdims=True)
        lane128 = lax.broadcasted_iota(jnp.int32, merged.shape, 1)
        lane = jnp.min(jnp.where(merged == fmin, lane128, jnp.int32(127)),
                       axis=1, keepdims=True)
        idp = lax.bitcast_convert_type(fmin, jnp.int32) & jnp.int32(_IDMASK)
        out_ref[...] = (idp * 128 + lane)[:, 0]


def _nn_argmin(inputs, keys):
    q, d = inputs.shape
    n = keys.shape[0]
    npad = ((n + _BN - 1) // _BN) * _BN
    kaug, qaug = _prep(inputs, keys, npad)
    grid = (q // _BQ, npad // _BN)
    return pl.pallas_call(
        functools.partial(_nn_body, bn=_BN),
        grid=grid,
        in_specs=[
            pl.BlockSpec((d + 4, _BQ), lambda iq, inn: (0, iq)),
            pl.BlockSpec((d + 4, _BN), lambda iq, inn: (0, inn)),
        ],
        out_specs=pl.BlockSpec((_BQ,), lambda iq, inn: (iq,)),
        out_shape=jax.ShapeDtypeStruct((q,), jnp.int32),
        scratch_shapes=[
            pltpu.VMEM((_BQ, 128), jnp.float32),
        ],
        compiler_params=pltpu.CompilerParams(
            dimension_semantics=("parallel", "arbitrary")),
    )(qaug, kaug)


def _gather_sigmoid(w_flat, idx):
    q = idx.shape[0]
    bpw = q // _NW
    mesh = plsc.VectorSubcoreMesh(core_axis_name="c", subcore_axis_name="s")

    @functools.partial(
        pl.kernel,
        out_type=jax.ShapeDtypeStruct((2, q), jnp.float32),
        mesh=mesh,
        scratch_types=[
            pltpu.VMEM((bpw,), jnp.int32),
            pltpu.VMEM((bpw,), jnp.float32),
            pltpu.VMEM((bpw,), jnp.float32),
            pltpu.VMEM((bpw,), jnp.float32),
            pltpu.SemaphoreType.DMA,
        ],
    )
    def k(w_hbm, idx_hbm, out_hbm, idx_v, val_v, y0_v, y1_v, sem):
        wid = lax.axis_index("s") * _NC + lax.axis_index("c")
        base = wid * bpw
        pltpu.sync_copy(idx_hbm.at[pl.ds(base, bpw)], idx_v)
        pltpu.async_copy(w_hbm.at[idx_v], val_v, sem).wait()
        for i in range(bpw // _L):
            x = val_v[pl.ds(i * _L, _L)]
            s = 1.0 / (1.0 + jnp.exp(-x))
            y0_v[pl.ds(i * _L, _L)] = s
            y1_v[pl.ds(i * _L, _L)] = 1.0 - s
        pltpu.sync_copy(y0_v, out_hbm.at[0, pl.ds(base, bpw)])
        pltpu.sync_copy(y1_v, out_hbm.at[1, pl.ds(base, bpw)])

    return k(w_flat, idx)


def kernel(inputs, keys, w):
    idx = _nn_argmin(inputs, keys)
    y01 = _gather_sigmoid(w.reshape(-1), idx)
    return y01.T
